# Initial kernel scaffold; baseline (speedup 1.0000x reference)
import jax
import jax.numpy as jnp
from jax.experimental import pallas as pl


def kernel(node_type_ids, adj, node_emb, type_emb, W1, b1, W2, b2):
    raise NotImplementedError("write your pallas kernel here")


# speedup vs baseline: 1.2055x; 1.2055x over previous
import functools

import jax
import jax.numpy as jnp
from jax.experimental import pallas as pl


def _encode_kernel(ids_ref, node_emb_ref, type_emb_ref, x_ref):
    ids = ids_ref[...]
    t = jax.lax.broadcasted_iota(jnp.int32, (ids.shape[0], type_emb_ref.shape[0]), 1)
    onehot = (ids == t).astype(jnp.float32)
    x = node_emb_ref[...] + jnp.dot(onehot, type_emb_ref[...],
                                    preferred_element_type=jnp.float32)
    x_ref[...] = x.astype(jnp.bfloat16)


def _gcn_kernel(adj_ref, x_ref, w_ref, b_ref, out_ref, *, relu, out_dtype):
    a_bf = adj_ref[...].astype(jnp.bfloat16)
    acc = jnp.dot(a_bf, x_ref[...], preferred_element_type=jnp.float32)
    y = jnp.dot(acc, w_ref[...], preferred_element_type=jnp.float32) + b_ref[...]
    if relu:
        y = jnp.maximum(y, 0.0)
    out_ref[...] = y.astype(out_dtype)


def _gcn_layer(adj, x, w, b, *, relu, out_dtype, block_m):
    n = adj.shape[0]
    k = x.shape[0]
    h_in = x.shape[1]
    h_out = w.shape[1]
    grid = (n // block_m,)
    return pl.pallas_call(
        functools.partial(_gcn_kernel, relu=relu, out_dtype=out_dtype),
        grid=grid,
        in_specs=[
            pl.BlockSpec((block_m, k), lambda i: (i, 0)),
            pl.BlockSpec((k, h_in), lambda i: (0, 0)),
            pl.BlockSpec((h_in, h_out), lambda i: (0, 0)),
            pl.BlockSpec((1, h_out), lambda i: (0, 0)),
        ],
        out_specs=pl.BlockSpec((block_m, h_out), lambda i: (i, 0)),
        out_shape=jax.ShapeDtypeStruct((n, h_out), out_dtype),
    )(adj, x, w, b)


def kernel(node_type_ids, adj, node_emb, type_emb, W1, b1, W2, b2):
    n, hidden = node_emb.shape
    num_types = type_emb.shape[0]
    embed = W2.shape[1]

    t_pad = 16
    type_emb_p = jnp.pad(type_emb, ((0, t_pad - num_types), (0, 0)))
    ids2 = node_type_ids.reshape(n, 1)

    x = pl.pallas_call(
        _encode_kernel,
        out_shape=jax.ShapeDtypeStruct((n, hidden), jnp.bfloat16),
    )(ids2, node_emb, type_emb_p)

    b1r = b1.reshape(1, hidden)
    b2r = b2.reshape(1, embed)

    block_m = 400
    h = _gcn_layer(adj, x, W1, b1r, relu=True, out_dtype=jnp.bfloat16,
                   block_m=block_m)
    z = _gcn_layer(adj, h, W2, b2r, relu=False, out_dtype=jnp.float32,
                   block_m=block_m)
    return z
